# baseline (speedup 1.0000x reference)
import functools

import jax
import jax.numpy as jnp
from jax import lax
from jax.experimental import pallas as pl
from jax.experimental.pallas import tpu as pltpu
from jax.experimental.pallas import tpu_sc as plsc

B = 16384
V = 100000
F = 116
D = 10
N = B * F

NC = 2
NS = 16
NW = NC * NS

G = 128
K = 16
GROUPS = N // G
GROUPS_PER_W = GROUPS // NW
STEPS = GROUPS_PER_W // K

@functools.cache
def _make_sc_gather():
    mesh = plsc.VectorSubcoreMesh(core_axis_name="c", subcore_axis_name="s")

    @functools.partial(
        pl.kernel,
        mesh=mesh,
        out_type=jax.ShapeDtypeStruct((N, D), jnp.float32),
        scratch_types=[
            pltpu.VMEM((K, G), jnp.int32),
            pltpu.VMEM((K * G, D), jnp.float32),
            pltpu.SemaphoreType.DMA,
        ],
        compiler_params=pltpu.CompilerParams(use_tc_tiling_on_sc=False),
    )
    def _sc_gather(idx_hbm, table_hbm, out_hbm, idx_v, rows_v, sem):
        wid = lax.axis_index("s") * NC + lax.axis_index("c")
        base_group = wid * GROUPS_PER_W

        def step(i, carry):
            g0 = base_group + i * K
            pltpu.sync_copy(idx_hbm.at[pl.ds(g0, K)], idx_v)
            copies = [
                pltpu.async_copy(
                    table_hbm.at[idx_v.at[j]], rows_v.at[pl.ds(j * G, G)], sem
                )
                for j in range(K)
            ]
            for c in copies:
                c.wait()
            pltpu.sync_copy(rows_v, out_hbm.at[pl.ds(g0 * G, K * G)])
            return carry

        lax.fori_loop(0, STEPS, step, 0)

    return _sc_gather


BLK = 1024


def _mlp_body(cont_ref, g_ref, w1_ref, b1_ref, w2a_ref, w2b_ref, b2_ref,
              w3_ref, b3_ref, out_ref):
    cont = cont_ref[...]
    h1 = lax.dot_general(cont, w1_ref[...], (((1,), (1,)), ((), ())),
                         preferred_element_type=jnp.float32)
    h1 = jnp.maximum(h1 + b1_ref[...], 0.0)
    g = jnp.maximum(g_ref[...], 0.0)
    h2 = lax.dot_general(h1, w2a_ref[...], (((1,), (1,)), ((), ())),
                         preferred_element_type=jnp.float32)
    h2 = h2 + lax.dot_general(g, w2b_ref[...], (((1,), (1,)), ((), ())),
                              preferred_element_type=jnp.float32)
    h2 = jnp.maximum(h2 + b2_ref[...], 0.0)
    out_ref[...] = lax.dot_general(h2, w3_ref[...], (((1,), (1,)), ((), ())),
                                   preferred_element_type=jnp.float32) + b3_ref[0, 0]


def _mlp(cont_x, gathered, W1, b1, W2a, W2b, b2, W3, b3):
    grid = (B // BLK,)
    return pl.pallas_call(
        _mlp_body,
        grid=grid,
        in_specs=[
            pl.BlockSpec((BLK, 14), lambda i: (i, 0)),
            pl.BlockSpec((BLK, F * D), lambda i: (i, 0)),
            pl.BlockSpec((32, 14), lambda i: (0, 0)),
            pl.BlockSpec((1, 32), lambda i: (0, 0)),
            pl.BlockSpec((128, 32), lambda i: (0, 0)),
            pl.BlockSpec((128, F * D), lambda i: (0, 0)),
            pl.BlockSpec((1, 128), lambda i: (0, 0)),
            pl.BlockSpec((128, 128), lambda i: (0, 0)),
            pl.BlockSpec((1, 1), lambda i: (0, 0)),
        ],
        out_specs=pl.BlockSpec((BLK, 128), lambda i: (i, 0)),
        out_shape=jax.ShapeDtypeStruct((B, 128), jnp.float32),
    )(cont_x, gathered, W1, b1, W2a, W2b, b2, W3, b3)


def kernel(cat_x, cont_x, emb, W1, b1, W2, b2, W3, b3):
    flat_idx = (cat_x.astype(jnp.int32)
                + (jnp.arange(F, dtype=jnp.int32) * V)[None, :]).reshape(GROUPS, G)
    table = emb.reshape(F * V, D)
    rows = _make_sc_gather()(flat_idx, table)
    gathered = rows.reshape(B, F * D)
    if True:
        h = jnp.concatenate([cont_x @ W1.T + b1, gathered], axis=1)
        h = jax.nn.relu(h)
        h = jax.nn.relu(h @ W2.T + b2)
        return h @ W3.T + b3
    W3p = jnp.zeros((128, 128), jnp.float32).at[:1, :].set(W3)
    out = _mlp(cont_x, gathered,
               W1, b1.reshape(1, 32),
               W2[:, :32], W2[:, 32:], b2.reshape(1, 128),
               W3p, b3.reshape(1, 1))
    return out[:, :1]


# speedup vs baseline: 1.0828x; 1.0828x over previous
import functools

import jax
import jax.numpy as jnp
from jax import lax
from jax.experimental import pallas as pl
from jax.experimental.pallas import tpu as pltpu
from jax.experimental.pallas import tpu_sc as plsc

B = 16384
V = 100000
F = 116
D = 10
DP = 16
N = B * F

NC = 2
NS = 16
NW = NC * NS

G = 128
K = 16
GROUPS = N // G
GROUPS_PER_W = GROUPS // NW
STEPS = GROUPS_PER_W // K


@functools.cache
def _make_sc_gather():
    mesh = plsc.VectorSubcoreMesh(core_axis_name="c", subcore_axis_name="s")

    @functools.partial(
        pl.kernel,
        mesh=mesh,
        out_type=jax.ShapeDtypeStruct((N, DP), jnp.float32),
        scratch_types=[
            pltpu.VMEM((K, G), jnp.int32),
            pltpu.VMEM((K * G, DP), jnp.float32),
            pltpu.SemaphoreType.DMA,
        ],
        compiler_params=pltpu.CompilerParams(use_tc_tiling_on_sc=False),
    )
    def _sc_gather(idx_hbm, table_hbm, out_hbm, idx_v, rows_v, sem):
        wid = lax.axis_index("s") * NC + lax.axis_index("c")
        base_group = wid * GROUPS_PER_W

        def step(i, carry):
            g0 = base_group + i * K
            pltpu.sync_copy(idx_hbm.at[pl.ds(g0, K)], idx_v)
            copies = [
                pltpu.async_copy(
                    table_hbm.at[idx_v.at[j]], rows_v.at[pl.ds(j * G, G)], sem
                )
                for j in range(K)
            ]
            for c in copies:
                c.wait()
            pltpu.sync_copy(rows_v, out_hbm.at[pl.ds(g0 * G, K * G)])
            return carry

        lax.fori_loop(0, STEPS, step, 0)

    return _sc_gather


BLK = 1024


def _mlp_body(cont_ref, g_ref, w1_ref, b1_ref, w2a_ref, w2b_ref, b2_ref,
              w3_ref, b3_ref, out_ref):
    cont = cont_ref[...]
    h1 = lax.dot_general(cont, w1_ref[...], (((1,), (1,)), ((), ())),
                         preferred_element_type=jnp.float32)
    h1 = jnp.maximum(h1 + b1_ref[...], 0.0)
    g = jnp.maximum(g_ref[...], 0.0)
    h2 = lax.dot_general(h1, w2a_ref[...], (((1,), (1,)), ((), ())),
                         preferred_element_type=jnp.float32)
    h2 = h2 + lax.dot_general(g, w2b_ref[...], (((1,), (1,)), ((), ())),
                              preferred_element_type=jnp.float32)
    h2 = jnp.maximum(h2 + b2_ref[...], 0.0)
    out_ref[...] = lax.dot_general(h2, w3_ref[...], (((1,), (1,)), ((), ())),
                                   preferred_element_type=jnp.float32) + b3_ref[0, 0]


def _mlp(cont_x, gathered, W1, b1, W2a, W2bp, b2, W3p, b3):
    grid = (B // BLK,)
    return pl.pallas_call(
        _mlp_body,
        grid=grid,
        in_specs=[
            pl.BlockSpec((BLK, 14), lambda i: (i, 0)),
            pl.BlockSpec((BLK, F * DP), lambda i: (i, 0)),
            pl.BlockSpec((32, 14), lambda i: (0, 0)),
            pl.BlockSpec((1, 32), lambda i: (0, 0)),
            pl.BlockSpec((128, 32), lambda i: (0, 0)),
            pl.BlockSpec((128, F * DP), lambda i: (0, 0)),
            pl.BlockSpec((1, 128), lambda i: (0, 0)),
            pl.BlockSpec((128, 128), lambda i: (0, 0)),
            pl.BlockSpec((1, 1), lambda i: (0, 0)),
        ],
        out_specs=pl.BlockSpec((BLK, 128), lambda i: (i, 0)),
        out_shape=jax.ShapeDtypeStruct((B, 128), jnp.float32),
    )(cont_x, gathered, W1, b1, W2a, W2bp, b2, W3p, b3)


def kernel(cat_x, cont_x, emb, W1, b1, W2, b2, W3, b3):
    flat_idx = (cat_x.astype(jnp.int32)
                + (jnp.arange(F, dtype=jnp.int32) * V)[None, :]).reshape(GROUPS, G)
    table = jnp.pad(emb, ((0, 0), (0, 0), (0, DP - D))).reshape(F * V, DP)
    rows = _make_sc_gather()(flat_idx, table)
    gathered = rows.reshape(B, F * DP)
    W2bp = jnp.pad(W2[:, 32:].reshape(128, F, D),
                   ((0, 0), (0, 0), (0, DP - D))).reshape(128, F * DP)
    W3p = jnp.zeros((128, 128), jnp.float32).at[:1, :].set(W3)
    out = _mlp(cont_x, gathered,
               W1, b1.reshape(1, 32),
               W2[:, :32], W2bp, b2.reshape(1, 128),
               W3p, b3.reshape(1, 1))
    return out[:, :1]
